# baseline (speedup 1.0000x reference)
import functools

import jax
import jax.numpy as jnp
from jax import lax
from jax.experimental import pallas as pl
from jax.experimental.pallas import tpu as pltpu
from jax.experimental.pallas import tpu_sc as plsc

B = 16384
SEQ = 50
PADS = 64
E = 32
NCLS = 20

_info = plsc.get_sparse_core_info()
NC, NS = _info.num_cores, _info.num_subcores
NW = NC * NS
RPW = B // NW
CHUNK_ROWS = 2
CHUNK_IDX = CHUNK_ROWS * PADS
NCHUNK = RPW // CHUNK_ROWS
IPW = RPW * PADS


def _tree_sum(loads):
    parts = []
    for k in range(4):
        chain = loads[k::4]
        acc = chain[0]
        for v in chain[1:]:
            acc = acc + v
        parts.append(acc)
    return (parts[0] + parts[1]) + (parts[2] + parts[3])


NBUF = 4
NGRP = NCHUNK // NBUF


def _sc_pool_body(ids_hbm, table_hbm, out_hbm, idx_v, rows, feat_v, sems):
    wid = lax.axis_index("s") * NC + lax.axis_index("c")
    base_row = wid * RPW
    base_idx = wid * IPW

    pltpu.sync_copy(ids_hbm.at[pl.ds(base_idx, IPW)], idx_v)

    def start(c, b):
        pltpu.async_copy(
            table_hbm.at[idx_v.at[pl.ds(c * CHUNK_IDX, CHUNK_IDX)]],
            rows[b], sems[b])

    def finish(c, b):
        pltpu.make_async_copy(
            table_hbm.at[idx_v.at[pl.ds(c * CHUNK_IDX, CHUNK_IDX)]],
            rows[b], sems[b]).wait()
        inv = jnp.float32(1.0 / SEQ)
        for r in range(CHUNK_ROWS):
            for h in range(2):
                loads = [rows[b][r * PADS + s, pl.ds(16 * h, 16)]
                         for s in range(PADS)]
                feat_v[c * CHUNK_ROWS + r, pl.ds(16 * h, 16)] = (
                    _tree_sum(loads) * inv)

    for b in range(NBUF):
        start(b, b)

    def group(g, _):
        for b in range(NBUF):
            c = g * NBUF + b
            finish(c, b)
            start(c + NBUF, b)
        return 0

    lax.fori_loop(0, NGRP - 1, group, 0)
    for b in range(NBUF):
        finish((NGRP - 1) * NBUF + b, b)

    pltpu.sync_copy(feat_v, out_hbm.at[pl.ds(base_row, RPW)])


@functools.partial(
    pl.kernel,
    out_type=jax.ShapeDtypeStruct((B, E), jnp.float32),
    mesh=plsc.VectorSubcoreMesh(core_axis_name="c", subcore_axis_name="s"),
    scratch_types=[
        pltpu.VMEM((IPW,), jnp.int32),
        [pltpu.VMEM((CHUNK_IDX, E), jnp.float32)
         for _ in range(NBUF)],
        pltpu.VMEM((RPW, E), jnp.float32),
        [pltpu.SemaphoreType.DMA for _ in range(NBUF)],
    ],
    compiler_params=pltpu.CompilerParams(use_tc_tiling_on_sc=False),
)
def _sc_pool(ids_hbm, table_hbm, out_hbm, idx_v, rows, feat_v, sems):
    _sc_pool_body(ids_hbm, table_hbm, out_hbm, idx_v, rows, feat_v, sems)


def _mm_body(f_ref, w_ref, b_ref, o_ref):
    o_ref[...] = (
        lax.dot_general(f_ref[...], w_ref[...],
                        (((1,), (1,)), ((), ())),
                        preferred_element_type=jnp.float32)
        + b_ref[...])


_MM_BLK = 1024


def _tc_logits(feats, W, b2d):
    return pl.pallas_call(
        _mm_body,
        grid=(B // _MM_BLK,),
        in_specs=[
            pl.BlockSpec((_MM_BLK, E), lambda i: (i, 0)),
            pl.BlockSpec((NCLS, E), lambda i: (0, 0)),
            pl.BlockSpec((1, NCLS), lambda i: (0, 0)),
        ],
        out_specs=pl.BlockSpec((_MM_BLK, NCLS), lambda i: (i, 0)),
        out_shape=jax.ShapeDtypeStruct((B, NCLS), jnp.float32),
    )(feats, W, b2d)


def kernel(input_ids, table, W, b):
    ids = input_ids.astype(jnp.int32)
    ids_pad = jnp.zeros((B, PADS), jnp.int32).at[:, :SEQ].set(ids)
    feats = _sc_pool(ids_pad.reshape(-1), table)
    return _tc_logits(feats, W, b.reshape(1, NCLS))


# speedup vs baseline: 4.5988x; 4.5988x over previous
import functools

import jax
import jax.numpy as jnp
from jax import lax
from jax.experimental import pallas as pl
from jax.experimental.pallas import tpu as pltpu
from jax.experimental.pallas import tpu_sc as plsc

B = 16384
SEQ = 50
PADS = 56
E = 32
NCLS = 20

_info = plsc.get_sparse_core_info()
NC, NS = _info.num_cores, _info.num_subcores
NW = NC * NS
RPW = B // NW
CHUNK_ROWS = 2
CHUNK_IDX = CHUNK_ROWS * PADS
NCHUNK = RPW // CHUNK_ROWS
IPW = RPW * PADS


def _tree_sum(loads):
    parts = []
    for k in range(4):
        chain = loads[k::4]
        acc = chain[0]
        for v in chain[1:]:
            acc = acc + v
        parts.append(acc)
    return (parts[0] + parts[1]) + (parts[2] + parts[3])


NBUF = 4
NGRP = NCHUNK // NBUF


def _sc_pool_body(ids_hbm, table_hbm, out_hbm, idx_v, rows, feat_v, sems):
    wid = lax.axis_index("s") * NC + lax.axis_index("c")
    base_row = wid * RPW
    base_idx = wid * IPW

    pltpu.sync_copy(ids_hbm.at[pl.ds(base_idx, IPW)], idx_v)

    def start(c, b):
        pltpu.async_copy(
            table_hbm.at[idx_v.at[pl.ds(c * CHUNK_IDX, CHUNK_IDX)]],
            rows[b], sems[b])

    def finish(c, b):
        pltpu.make_async_copy(
            table_hbm.at[idx_v.at[pl.ds(c * CHUNK_IDX, CHUNK_IDX)]],
            rows[b], sems[b]).wait()
        inv = jnp.float32(1.0 / SEQ)
        for r in range(CHUNK_ROWS):
            for h in range(2):
                loads = [rows[b][r * PADS + s, pl.ds(16 * h, 16)]
                         for s in range(SEQ)]
                feat_v[c * CHUNK_ROWS + r, pl.ds(16 * h, 16)] = (
                    _tree_sum(loads) * inv)

    for b in range(NBUF):
        start(b, b)

    def group(g, _):
        for b in range(NBUF):
            c = g * NBUF + b
            finish(c, b)
            start(c + NBUF, b)
        return 0

    lax.fori_loop(0, NGRP - 1, group, 0)
    for b in range(NBUF):
        finish((NGRP - 1) * NBUF + b, b)

    pltpu.sync_copy(feat_v, out_hbm.at[pl.ds(base_row, RPW)])


@functools.partial(
    pl.kernel,
    out_type=jax.ShapeDtypeStruct((B, E), jnp.float32),
    mesh=plsc.VectorSubcoreMesh(core_axis_name="c", subcore_axis_name="s"),
    scratch_types=[
        pltpu.VMEM((IPW,), jnp.int32),
        [pltpu.VMEM((CHUNK_IDX, E), jnp.float32)
         for _ in range(NBUF)],
        pltpu.VMEM((RPW, E), jnp.float32),
        [pltpu.SemaphoreType.DMA for _ in range(NBUF)],
    ],
    compiler_params=pltpu.CompilerParams(use_tc_tiling_on_sc=False),
)
def _sc_pool(ids_hbm, table_hbm, out_hbm, idx_v, rows, feat_v, sems):
    _sc_pool_body(ids_hbm, table_hbm, out_hbm, idx_v, rows, feat_v, sems)


def _mm_body(f_ref, w_ref, b_ref, o_ref):
    o_ref[...] = (
        lax.dot_general(f_ref[...], w_ref[...],
                        (((1,), (1,)), ((), ())),
                        preferred_element_type=jnp.float32)
        + b_ref[...])


_MM_BLK = 1024


def _tc_logits(feats, W, b2d):
    return pl.pallas_call(
        _mm_body,
        grid=(B // _MM_BLK,),
        in_specs=[
            pl.BlockSpec((_MM_BLK, E), lambda i: (i, 0)),
            pl.BlockSpec((NCLS, E), lambda i: (0, 0)),
            pl.BlockSpec((1, NCLS), lambda i: (0, 0)),
        ],
        out_specs=pl.BlockSpec((_MM_BLK, NCLS), lambda i: (i, 0)),
        out_shape=jax.ShapeDtypeStruct((B, NCLS), jnp.float32),
    )(feats, W, b2d)


def kernel(input_ids, table, W, b):
    ids = input_ids.astype(jnp.int32)
    ids_pad = jnp.concatenate([ids, ids[:, :PADS - SEQ]], axis=1)
    feats = _sc_pool(ids_pad.reshape(-1), table)
    return _tc_logits(feats, W, b.reshape(1, NCLS))


# speedup vs baseline: 4.6187x; 1.0043x over previous
import functools

import jax
import jax.numpy as jnp
from jax import lax
from jax.experimental import pallas as pl
from jax.experimental.pallas import tpu as pltpu
from jax.experimental.pallas import tpu_sc as plsc

B = 16384
SEQ = 50
PADS = 56
E = 32
NCLS = 20
VOCAB = 1000000

_info = plsc.get_sparse_core_info()
NC, NS = _info.num_cores, _info.num_subcores
NW = NC * NS
RPW = B // NW
CHUNK_ROWS = 2
CHUNK_IDX = CHUNK_ROWS * PADS
NCHUNK = RPW // CHUNK_ROWS
IPW = RPW * PADS


def _tree_sum(loads):
    parts = []
    for k in range(4):
        chain = loads[k::4]
        acc = chain[0]
        for v in chain[1:]:
            acc = acc + v
        parts.append(acc)
    return (parts[0] + parts[1]) + (parts[2] + parts[3])


NBUF = 4
NGRP = NCHUNK // NBUF


def _sc_pool_body(ids_hbm, table_hbm, out_hbm, idx_v, rows, feat_v, sems):
    wid = lax.axis_index("s") * NC + lax.axis_index("c")
    base_row = wid * RPW
    base_idx = wid * IPW

    pltpu.sync_copy(ids_hbm.at[pl.ds(base_idx, IPW)], idx_v)

    tbl = table_hbm

    def start(c, b):
        pltpu.async_copy(
            tbl.at[idx_v.at[pl.ds(c * CHUNK_IDX, CHUNK_IDX)]],
            rows[b], sems[b])

    def finish(c, b):
        pltpu.make_async_copy(
            tbl.at[idx_v.at[pl.ds(c * CHUNK_IDX, CHUNK_IDX)]],
            rows[b], sems[b]).wait()
        inv = jnp.float32(1.0 / SEQ)
        for r in range(CHUNK_ROWS):
            for h in range(2):
                loads = [rows[b][r * PADS + s, pl.ds(16 * h, 16)]
                         for s in range(SEQ)]
                feat_v[c * CHUNK_ROWS + r, pl.ds(16 * h, 16)] = (
                    _tree_sum(loads) * inv)

    for b in range(NBUF):
        start(b, b)

    def group(g, _):
        for b in range(NBUF):
            c = g * NBUF + b
            finish(c, b)
            start(c + NBUF, b)
        return 0

    lax.fori_loop(0, NGRP - 1, group, 0)
    for b in range(NBUF):
        finish((NGRP - 1) * NBUF + b, b)

    pltpu.sync_copy(feat_v, out_hbm.at[pl.ds(base_row, RPW)])


@functools.partial(
    pl.kernel,
    out_type=jax.ShapeDtypeStruct((B, E), jnp.float32),
    mesh=plsc.VectorSubcoreMesh(core_axis_name="c", subcore_axis_name="s"),
    scratch_types=[
        pltpu.VMEM((IPW,), jnp.int32),
        [pltpu.VMEM((CHUNK_IDX, E), jnp.float32)
         for _ in range(NBUF)],
        pltpu.VMEM((RPW, E), jnp.float32),
        [pltpu.SemaphoreType.DMA for _ in range(NBUF)],
    ],
    compiler_params=pltpu.CompilerParams(use_tc_tiling_on_sc=False),
)
def _sc_pool(ids_hbm, table_hbm, out_hbm, idx_v, rows, feat_v, sems):
    _sc_pool_body(ids_hbm, table_hbm, out_hbm, idx_v, rows, feat_v, sems)


_TR_BK = 2048
_TR_OUT = _TR_BK // 4


def _tr_body(t_ref, o_ref):
    x = t_ref[...]
    y = x.T.reshape(_TR_OUT, 4, E)
    o_ref[...] = jnp.concatenate([y[:, a, :] for a in range(4)], axis=1)


def _tc_transpose(tableT):
    return pl.pallas_call(
        _tr_body,
        grid=(pl.cdiv(VOCAB, _TR_BK),),
        in_specs=[pl.BlockSpec((E, _TR_BK), lambda i: (0, i))],
        out_specs=pl.BlockSpec((_TR_OUT, 4 * E), lambda i: (i, 0)),
        out_shape=jax.ShapeDtypeStruct((VOCAB // 4, 4 * E), jnp.float32),
    )(tableT)


def _mm_body(f_ref, w_ref, b_ref, o_ref):
    o_ref[...] = (
        lax.dot_general(f_ref[...], w_ref[...],
                        (((1,), (1,)), ((), ())),
                        preferred_element_type=jnp.float32)
        + b_ref[...])


_MM_BLK = 1024


def _tc_logits(feats, W, b2d):
    return pl.pallas_call(
        _mm_body,
        grid=(B // _MM_BLK,),
        in_specs=[
            pl.BlockSpec((_MM_BLK, E), lambda i: (i, 0)),
            pl.BlockSpec((NCLS, E), lambda i: (0, 0)),
            pl.BlockSpec((1, NCLS), lambda i: (0, 0)),
        ],
        out_specs=pl.BlockSpec((_MM_BLK, NCLS), lambda i: (i, 0)),
        out_shape=jax.ShapeDtypeStruct((B, NCLS), jnp.float32),
    )(feats, W, b2d)


def kernel(input_ids, table, W, b):
    ids = input_ids.astype(jnp.int32)
    ids_pad = jnp.concatenate([ids, ids[:, :PADS - SEQ]], axis=1)
    t_lin = _tc_transpose(table.T)
    feats = _sc_pool(ids_pad.reshape(-1), t_lin.reshape(VOCAB, E))
    return _tc_logits(feats, W, b.reshape(1, NCLS))


# speedup vs baseline: 5.3914x; 1.1673x over previous
import functools

import jax
import jax.numpy as jnp
from jax import lax
from jax.experimental import pallas as pl
from jax.experimental.pallas import tpu as pltpu
from jax.experimental.pallas import tpu_sc as plsc

B = 16384
SEQ = 50
PADS = 56
E = 32
NCLS = 20
VOCAB = 1000000

_info = plsc.get_sparse_core_info()
NC, NS = _info.num_cores, _info.num_subcores
NW = NC * NS
RPW = B // NW
CHUNK_ROWS = 2
CHUNK_IDX = CHUNK_ROWS * PADS
NCHUNK = RPW // CHUNK_ROWS
IPW = RPW * PADS


def _tree_sum(loads):
    parts = []
    for k in range(4):
        chain = loads[k::4]
        acc = chain[0]
        for v in chain[1:]:
            acc = acc + v
        parts.append(acc)
    return (parts[0] + parts[1]) + (parts[2] + parts[3])


NBUF = 4
NGRP = NCHUNK // NBUF


def _sc_pool_body(ids_hbm, table_hbm, out_hbm, idx_v, rows, feat_v, sems):
    wid = lax.axis_index("s") * NC + lax.axis_index("c")
    base_row = wid * RPW
    base_idx = wid * IPW

    pltpu.sync_copy(ids_hbm.at[pl.ds(base_idx, IPW)], idx_v)

    tbl = table_hbm

    def start(c, b):
        pltpu.async_copy(
            tbl.at[idx_v.at[pl.ds(c * CHUNK_IDX, CHUNK_IDX)]],
            rows[b], sems[b])

    def finish(c, b):
        pltpu.make_async_copy(
            tbl.at[idx_v.at[pl.ds(c * CHUNK_IDX, CHUNK_IDX)]],
            rows[b], sems[b]).wait()
        inv = jnp.float32(1.0 / SEQ)
        for r in range(CHUNK_ROWS):
            for h in range(2):
                loads = [rows[b][r * PADS + s, pl.ds(16 * h, 16)]
                         for s in range(SEQ)]
                feat_v[c * CHUNK_ROWS + r, pl.ds(16 * h, 16)] = (
                    _tree_sum(loads) * inv)

    for b in range(NBUF):
        start(b, b)

    def group(g, _):
        for b in range(NBUF):
            c = g * NBUF + b
            finish(c, b)
            start(c + NBUF, b)
        return 0

    lax.fori_loop(0, NGRP - 1, group, 0)
    for b in range(NBUF):
        finish((NGRP - 1) * NBUF + b, b)

    pltpu.sync_copy(feat_v, out_hbm.at[pl.ds(base_row, RPW)])


@functools.partial(
    pl.kernel,
    out_type=jax.ShapeDtypeStruct((B, E), jnp.float32),
    mesh=plsc.VectorSubcoreMesh(core_axis_name="c", subcore_axis_name="s"),
    scratch_types=[
        pltpu.VMEM((IPW,), jnp.int32),
        [pltpu.VMEM((CHUNK_IDX, E), jnp.float32)
         for _ in range(NBUF)],
        pltpu.VMEM((RPW, E), jnp.float32),
        [pltpu.SemaphoreType.DMA for _ in range(NBUF)],
    ],
    compiler_params=pltpu.CompilerParams(use_tc_tiling_on_sc=False),
)
def _sc_pool(ids_hbm, table_hbm, out_hbm, idx_v, rows, feat_v, sems):
    _sc_pool_body(ids_hbm, table_hbm, out_hbm, idx_v, rows, feat_v, sems)


_TR_BK = 8192
_TR_OUT = _TR_BK // 4


def _tr_body(t_ref, o_ref):
    x = t_ref[...]
    y = x.T.reshape(_TR_OUT, 4, E)
    o_ref[...] = jnp.concatenate([y[:, a, :] for a in range(4)], axis=1)


def _tc_transpose(tableT):
    return pl.pallas_call(
        _tr_body,
        grid=(pl.cdiv(VOCAB, _TR_BK),),
        in_specs=[pl.BlockSpec((E, _TR_BK), lambda i: (0, i))],
        out_specs=pl.BlockSpec((_TR_OUT, 4 * E), lambda i: (i, 0)),
        out_shape=jax.ShapeDtypeStruct((VOCAB // 4, 4 * E), jnp.float32),
    )(tableT)


def _mm_body(f_ref, w_ref, b_ref, o_ref):
    o_ref[...] = (
        lax.dot_general(f_ref[...], w_ref[...],
                        (((1,), (1,)), ((), ())),
                        preferred_element_type=jnp.float32)
        + b_ref[...])


_MM_BLK = 1024


def _tc_logits(feats, W, b2d):
    return pl.pallas_call(
        _mm_body,
        grid=(B // _MM_BLK,),
        in_specs=[
            pl.BlockSpec((_MM_BLK, E), lambda i: (i, 0)),
            pl.BlockSpec((NCLS, E), lambda i: (0, 0)),
            pl.BlockSpec((1, NCLS), lambda i: (0, 0)),
        ],
        out_specs=pl.BlockSpec((_MM_BLK, NCLS), lambda i: (i, 0)),
        out_shape=jax.ShapeDtypeStruct((B, NCLS), jnp.float32),
    )(feats, W, b2d)


def kernel(input_ids, table, W, b):
    ids = input_ids.astype(jnp.int32)
    ids_pad = jnp.concatenate([ids, ids[:, :PADS - SEQ]], axis=1)
    t_lin = _tc_transpose(table.T)
    feats = _sc_pool(ids_pad.reshape(-1), t_lin.reshape(VOCAB, E))
    return _tc_logits(feats, W, b.reshape(1, NCLS))


# speedup vs baseline: 5.4521x; 1.0113x over previous
import functools

import jax
import jax.numpy as jnp
from jax import lax
from jax.experimental import pallas as pl
from jax.experimental.pallas import tpu as pltpu
from jax.experimental.pallas import tpu_sc as plsc

B = 16384
SEQ = 50
PADS = 56
E = 32
NCLS = 20
VOCAB = 1000000

_info = plsc.get_sparse_core_info()
NC, NS = _info.num_cores, _info.num_subcores
NW = NC * NS
RPW = B // NW
CHUNK_ROWS = 2
CHUNK_IDX = CHUNK_ROWS * PADS
NCHUNK = RPW // CHUNK_ROWS
IPW = RPW * PADS


def _tree_sum(loads):
    parts = []
    for k in range(4):
        chain = loads[k::4]
        acc = chain[0]
        for v in chain[1:]:
            acc = acc + v
        parts.append(acc)
    return (parts[0] + parts[1]) + (parts[2] + parts[3])


NBUF = 4
NGRP = NCHUNK // NBUF


def _sc_pool_body(ids_hbm, table_hbm, out_hbm, idx_v, rows, feat_v, sems):
    wid = lax.axis_index("s") * NC + lax.axis_index("c")
    base_row = wid * RPW
    base_idx = wid * IPW

    pltpu.sync_copy(ids_hbm.at[pl.ds(base_idx, IPW)], idx_v)

    tbl = table_hbm

    def start(c, b):
        pltpu.async_copy(
            tbl.at[idx_v.at[pl.ds(c * CHUNK_IDX, CHUNK_IDX)]],
            rows[b], sems[b])

    def finish(c, b):
        pltpu.make_async_copy(
            tbl.at[idx_v.at[pl.ds(c * CHUNK_IDX, CHUNK_IDX)]],
            rows[b], sems[b]).wait()
        inv = jnp.float32(1.0 / SEQ)
        for r in range(CHUNK_ROWS):
            for h in range(2):
                loads = [rows[b][r * PADS + s, pl.ds(16 * h, 16)]
                         for s in range(SEQ)]
                feat_v[c * CHUNK_ROWS + r, pl.ds(16 * h, 16)] = (
                    _tree_sum(loads) * inv)

    for b in range(NBUF):
        start(b, b)

    def group(g, _):
        for b in range(NBUF):
            c = g * NBUF + b
            finish(c, b)
            start(c + NBUF, b)
        return 0

    lax.fori_loop(0, NGRP - 1, group, 0)
    for b in range(NBUF):
        finish((NGRP - 1) * NBUF + b, b)

    pltpu.sync_copy(feat_v, out_hbm.at[pl.ds(base_row, RPW)])


@functools.partial(
    pl.kernel,
    out_type=jax.ShapeDtypeStruct((B, E), jnp.float32),
    mesh=plsc.VectorSubcoreMesh(core_axis_name="c", subcore_axis_name="s"),
    scratch_types=[
        pltpu.VMEM((IPW,), jnp.int32),
        [pltpu.VMEM((CHUNK_IDX, E), jnp.float32)
         for _ in range(NBUF)],
        pltpu.VMEM((RPW, E), jnp.float32),
        [pltpu.SemaphoreType.DMA for _ in range(NBUF)],
    ],
    compiler_params=pltpu.CompilerParams(use_tc_tiling_on_sc=False),
)
def _sc_pool(ids_hbm, table_hbm, out_hbm, idx_v, rows, feat_v, sems):
    _sc_pool_body(ids_hbm, table_hbm, out_hbm, idx_v, rows, feat_v, sems)


_TR_BK = 16384
_TR_OUT = _TR_BK // 4


def _tr_body(t_ref, o_ref):
    x = t_ref[...]
    y = x.T.reshape(_TR_OUT, 4, E)
    o_ref[...] = jnp.concatenate([y[:, a, :] for a in range(4)], axis=1)


def _tc_transpose(tableT):
    return pl.pallas_call(
        _tr_body,
        grid=(pl.cdiv(VOCAB, _TR_BK),),
        in_specs=[pl.BlockSpec((E, _TR_BK), lambda i: (0, i))],
        out_specs=pl.BlockSpec((_TR_OUT, 4 * E), lambda i: (i, 0)),
        out_shape=jax.ShapeDtypeStruct((VOCAB // 4, 4 * E), jnp.float32),
    )(tableT)


def _mm_body(f_ref, w_ref, b_ref, o_ref):
    o_ref[...] = (
        lax.dot_general(f_ref[...], w_ref[...],
                        (((1,), (1,)), ((), ())),
                        preferred_element_type=jnp.float32)
        + b_ref[...])


_MM_BLK = 1024


def _tc_logits(feats, W, b2d):
    return pl.pallas_call(
        _mm_body,
        grid=(B // _MM_BLK,),
        in_specs=[
            pl.BlockSpec((_MM_BLK, E), lambda i: (i, 0)),
            pl.BlockSpec((NCLS, E), lambda i: (0, 0)),
            pl.BlockSpec((1, NCLS), lambda i: (0, 0)),
        ],
        out_specs=pl.BlockSpec((_MM_BLK, NCLS), lambda i: (i, 0)),
        out_shape=jax.ShapeDtypeStruct((B, NCLS), jnp.float32),
    )(feats, W, b2d)


def kernel(input_ids, table, W, b):
    ids = input_ids.astype(jnp.int32)
    ids_pad = jnp.concatenate([ids, ids[:, :PADS - SEQ]], axis=1)
    t_lin = _tc_transpose(table.T)
    feats = _sc_pool(ids_pad.reshape(-1), t_lin.reshape(VOCAB, E))
    return _tc_logits(feats, W, b.reshape(1, NCLS))


# speedup vs baseline: 11.9826x; 2.1978x over previous
import functools

import jax
import jax.numpy as jnp
from jax import lax
from jax.experimental import pallas as pl
from jax.experimental.pallas import tpu as pltpu
from jax.experimental.pallas import tpu_sc as plsc

B = 16384
SEQ = 50
PADS = 56
E = 32
NCLS = 20
VOCAB = 1000000

_info = plsc.get_sparse_core_info()
NC, NS = _info.num_cores, _info.num_subcores
NW = NC * NS
RPW = B // NW
CHUNK_ROWS = 2
CHUNK_IDX = CHUNK_ROWS * PADS
NCHUNK = RPW // CHUNK_ROWS
IPW = RPW * PADS


def _tree_sum(loads):
    parts = []
    for k in range(4):
        chain = loads[k::4]
        acc = chain[0]
        for v in chain[1:]:
            acc = acc + v
        parts.append(acc)
    return (parts[0] + parts[1]) + (parts[2] + parts[3])


NBUF = 4
NGRP = NCHUNK // NBUF


def _sc_pool_body(ids_hbm, table_hbm, out_hbm, idx_v, rows, feat_v, sems):
    wid = lax.axis_index("s") * NC + lax.axis_index("c")
    base_row = wid * RPW
    base_idx = wid * IPW

    pltpu.sync_copy(ids_hbm.at[pl.ds(base_idx, IPW)], idx_v)

    tbl = table_hbm

    def start(c, b):
        pltpu.async_copy(
            tbl.at[idx_v.at[pl.ds(c * CHUNK_IDX, CHUNK_IDX)]],
            rows[b], sems[b])

    def finish(c, b):
        pltpu.make_async_copy(
            tbl.at[idx_v.at[pl.ds(c * CHUNK_IDX, CHUNK_IDX)]],
            rows[b], sems[b]).wait()
        inv = jnp.float32(1.0 / SEQ)
        for r in range(CHUNK_ROWS):
            for h in range(2):
                loads = [rows[b][r * PADS + s, pl.ds(16 * h, 16)]
                         for s in range(SEQ)]
                feat_v[c * CHUNK_ROWS + r, pl.ds(16 * h, 16)] = (
                    _tree_sum(loads) * inv)

    for b in range(NBUF):
        start(b, b)

    def group(g, _):
        for b in range(NBUF):
            c = g * NBUF + b
            finish(c, b)
            start(c + NBUF, b)
        return 0

    lax.fori_loop(0, NGRP - 1, group, 0)
    for b in range(NBUF):
        finish((NGRP - 1) * NBUF + b, b)

    pltpu.sync_copy(feat_v, out_hbm.at[pl.ds(base_row, RPW)])


@functools.partial(
    pl.kernel,
    out_type=jax.ShapeDtypeStruct((B, E), jnp.float32),
    mesh=plsc.VectorSubcoreMesh(core_axis_name="c", subcore_axis_name="s"),
    scratch_types=[
        pltpu.VMEM((IPW,), jnp.int32),
        [pltpu.VMEM((CHUNK_IDX, E), jnp.float32)
         for _ in range(NBUF)],
        pltpu.VMEM((RPW, E), jnp.float32),
        [pltpu.SemaphoreType.DMA for _ in range(NBUF)],
    ],
    compiler_params=pltpu.CompilerParams(use_tc_tiling_on_sc=False),
)
def _sc_pool(ids_hbm, table_hbm, out_hbm, idx_v, rows, feat_v, sems):
    _sc_pool_body(ids_hbm, table_hbm, out_hbm, idx_v, rows, feat_v, sems)


_TR_BK = 16384
_TR_OUT = _TR_BK // 4
_NCHIP = _TR_BK // 512
VOCAB_PAD = ((VOCAB + 511) // 512) * 512


def _tr_body(t_ref, o_ref):
    x = t_ref[...]
    for j in range(_NCHIP):
        xq = jnp.concatenate(
            [x[:, 512 * j + 128 * q: 512 * j + 128 * (q + 1)]
             for q in range(4)], axis=0)
        o_ref[pl.ds(128 * j, 128), :] = xq.T


def _tc_transpose(tableT):
    return pl.pallas_call(
        _tr_body,
        grid=(pl.cdiv(VOCAB, _TR_BK),),
        in_specs=[pl.BlockSpec((E, _TR_BK), lambda i: (0, i))],
        out_specs=pl.BlockSpec((_TR_OUT, 4 * E), lambda i: (i, 0)),
        out_shape=jax.ShapeDtypeStruct((VOCAB_PAD // 4, 4 * E), jnp.float32),
    )(tableT)


def _mm_body(f_ref, w_ref, b_ref, o_ref):
    o_ref[...] = (
        lax.dot_general(f_ref[...], w_ref[...],
                        (((1,), (1,)), ((), ())),
                        preferred_element_type=jnp.float32)
        + b_ref[...])


_MM_BLK = 1024


def _tc_logits(feats, W, b2d):
    return pl.pallas_call(
        _mm_body,
        grid=(B // _MM_BLK,),
        in_specs=[
            pl.BlockSpec((_MM_BLK, E), lambda i: (i, 0)),
            pl.BlockSpec((NCLS, E), lambda i: (0, 0)),
            pl.BlockSpec((1, NCLS), lambda i: (0, 0)),
        ],
        out_specs=pl.BlockSpec((_MM_BLK, NCLS), lambda i: (i, 0)),
        out_shape=jax.ShapeDtypeStruct((B, NCLS), jnp.float32),
    )(feats, W, b2d)


def kernel(input_ids, table, W, b):
    ids = input_ids.astype(jnp.int32)
    ids_pad = jnp.concatenate([ids, ids[:, :PADS - SEQ]], axis=1)
    ids_sig = (ids_pad - ids_pad % 512 + 4 * (ids_pad % 128)
               + (ids_pad % 512) // 128)
    t_lin = _tc_transpose(table.T)
    feats = _sc_pool(ids_sig.reshape(-1), t_lin.reshape(VOCAB_PAD, E))
    return _tc_logits(feats, W, b.reshape(1, NCLS))
